# baseline (speedup 1.0000x reference)
import functools
import math

import jax
import jax.numpy as jnp
from jax.experimental import pallas as pl
from jax.experimental.pallas import tpu as pltpu

_K = 64
_CB = 4
_NC = 8


def _attn_manual_kernel(x_hbm, o_hbm, xbuf, obuf, lsem, ssem, *, scale):
    def load(c):
        return pltpu.make_async_copy(x_hbm.at[c], xbuf.at[c], lsem.at[c])

    def store(c):
        return pltpu.make_async_copy(obuf.at[c], o_hbm.at[c], ssem.at[c])

    for c in range(_NC):
        load(c).start()
    for c in range(_NC):
        load(c).wait()
        xh = xbuf[c].astype(jnp.bfloat16)
        s = jax.lax.dot_general(
            xh, xh, (((2,), (2,)), ((0,), (0,))),
            preferred_element_type=jnp.float32) * scale
        n = xh.shape[1]
        row = jax.lax.broadcasted_iota(jnp.int32, (1, n, n), 1)
        col = jax.lax.broadcasted_iota(jnp.int32, (1, n, n), 2)
        s = jnp.where(col <= row, s, -jnp.inf)
        m = jnp.max(s, axis=2, keepdims=True)
        p = jnp.exp(s - m)
        z = jnp.sum(p, axis=2, keepdims=True)
        p = (p / z).astype(jnp.bfloat16)
        obuf[c] = jax.lax.dot_general(
            p, xh, (((2,), (1,)), ((0,), (0,))),
            preferred_element_type=jnp.float32)
        store(c).start()
    for c in range(_NC):
        store(c).wait()


def kernel(x):
    b, t, e = x.shape
    scale = 1.0 / math.sqrt(e)
    nblocks = t // _K
    assert nblocks == _CB * _NC
    x2 = x.reshape(_NC, _CB, _K, e)
    out = pl.pallas_call(
        functools.partial(_attn_manual_kernel, scale=scale),
        in_specs=[pl.BlockSpec(memory_space=pl.ANY)],
        out_specs=pl.BlockSpec(memory_space=pl.ANY),
        out_shape=jax.ShapeDtypeStruct((_NC, _CB, _K, e), jnp.float32),
        scratch_shapes=[
            pltpu.VMEM((_NC, _CB, _K, e), jnp.float32),
            pltpu.VMEM((_NC, _CB, _K, e), jnp.float32),
            pltpu.SemaphoreType.DMA((_NC,)),
            pltpu.SemaphoreType.DMA((_NC,)),
        ],
    )(x2)
    return out.reshape(b, t, e)


# speedup vs baseline: 1.2166x; 1.2166x over previous
import functools
import math

import jax
import jax.numpy as jnp
from jax.experimental import pallas as pl
from jax.experimental.pallas import tpu as pltpu

_K = 64


def _attn_block_kernel(x_ref, o_ref, *, scale):
    xb = x_ref[...]
    s = jax.lax.dot_general(
        xb, xb, (((2,), (2,)), ((0,), (0,))),
        preferred_element_type=jnp.float32) * scale
    n = xb.shape[1]
    row = jax.lax.broadcasted_iota(jnp.int32, (1, n, n), 1)
    col = jax.lax.broadcasted_iota(jnp.int32, (1, n, n), 2)
    s = jnp.where(col <= row, s, -jnp.inf)
    m = jnp.max(s, axis=2, keepdims=True)
    p = jnp.exp(s - m)
    z = jnp.sum(p, axis=2, keepdims=True)
    p = p / z
    o_ref[...] = jax.lax.dot_general(
        p, xb, (((2,), (1,)), ((0,), (0,))),
        preferred_element_type=jnp.float32)


def kernel(x):
    b, t, e = x.shape
    scale = 1.0 / math.sqrt(e)
    nblocks = t // _K
    bp = min(16, nblocks)
    grid = (nblocks // bp,)
    x2 = x.reshape(nblocks, _K, e)
    out = pl.pallas_call(
        functools.partial(_attn_block_kernel, scale=scale),
        grid=grid,
        in_specs=[pl.BlockSpec((bp, _K, e), lambda i: (i, 0, 0))],
        out_specs=pl.BlockSpec((bp, _K, e), lambda i: (i, 0, 0)),
        out_shape=jax.ShapeDtypeStruct((nblocks, _K, e), jnp.float32),
        compiler_params=pltpu.CompilerParams(
            dimension_semantics=("arbitrary",),
        ),
    )(x2)
    return out.reshape(b, t, e)
